# Initial kernel scaffold; baseline (speedup 1.0000x reference)
import jax
import jax.numpy as jnp
from jax.experimental import pallas as pl


def kernel(x, c1_w, c1_b, c2_w, c2_b, c3_w, c3_b, c4_w, c4_b, c5_w, c5_b, l1_w, l1_b, l2_w, l2_b, l3_w, l3_b):
    raise NotImplementedError("write your pallas kernel here")


# speedup vs baseline: 1.0909x; 1.0909x over previous
import jax
import jax.numpy as jnp
from jax.experimental import pallas as pl
from jax.experimental.pallas import tpu as pltpu

IB = 32
N_IMG = 2048


def _even_cols_selector(pw, ow):
    r = jax.lax.broadcasted_iota(jnp.int32, (pw, ow), 0)
    c = jax.lax.broadcasted_iota(jnp.int32, (pw, ow), 1)
    return (c == 2 * r).astype(jnp.float32)


def _conv5x5(xpad, xcat, w_ref, *, cin, wp8, m_out):
    r_cat = xcat.shape[0]
    for j in range(5):
        xcat[:, j * cin:(j + 1) * cin] = xpad[pl.ds(j, r_cat), :]
    acc = None
    for i in range(5):
        part = jnp.dot(xcat[pl.ds(i * wp8, m_out), :], w_ref[i],
                       preferred_element_type=jnp.float32)
        acc = part if acc is None else acc + part
    return acc


def _pool_store(acc, b_ref, dst, *, wp8, ow, ph, pw, row0, stride, sel):
    bias = b_ref[...]
    for py in range(ph):
        r0 = 2 * py * wp8
        m = jnp.maximum(
            jnp.maximum(acc[pl.ds(r0, ow), :], acc[pl.ds(r0 + 1, ow), :]),
            jnp.maximum(acc[pl.ds(r0 + wp8, ow), :],
                        acc[pl.ds(r0 + wp8 + 1, ow), :]))
        p = jnp.dot(sel, m, preferred_element_type=jnp.float32)
        p = jnp.maximum(p + bias, 0.0).astype(dst.dtype)
        dst[pl.ds(row0 + py * stride, pw), :] = p


def _relu_store(acc, b_ref, dst, *, wp8, oh, ow, row0, stride):
    bias = b_ref[...]
    for y in range(oh):
        v = jnp.maximum(acc[pl.ds(y * wp8, ow), :] + bias, 0.0)
        dst[pl.ds(row0 + y * stride, ow), :] = v.astype(dst.dtype)


def _fused_kernel(x_ref, w1, b1, w2, b2, w3, b3, w4, b4, w5, b5,
                  wl1, bl1, wl2, bl2, wl3, bl3, out_ref,
                  xpad1, xcat1, acc1, xpad2, xcat2, acc2,
                  xpad3, xcat3, acc3, xpad4, xcat4, acc4,
                  xpad5, xcat5, feat):
    xpad1[...] = jnp.zeros_like(xpad1)
    xpad2[...] = jnp.zeros_like(xpad2)
    xpad3[...] = jnp.zeros_like(xpad3)
    xpad4[...] = jnp.zeros_like(xpad4)
    xpad5[...] = jnp.zeros_like(xpad5)
    feat[...] = jnp.zeros_like(feat)

    sel1 = _even_cols_selector(16, 32)
    sel2 = _even_cols_selector(8, 16)

    def one_image(i, carry):
        base = i * 1024
        for y in range(32):
            xpad1[pl.ds((2 + y) * 40 + 2, 32), :] = \
                x_ref[pl.ds(base + y * 32, 32), :]

        acc1[...] = _conv5x5(xpad1, xcat1, w1, cin=8, wp8=40, m_out=1280)
        _pool_store(acc1, b1, xpad2, wp8=40, ow=32, ph=16, pw=16,
                    row0=2 * 24 + 2, stride=24, sel=sel1)

        acc2[...] = _conv5x5(xpad2, xcat2, w2, cin=128, wp8=24, m_out=384)
        _pool_store(acc2, b2, xpad3, wp8=24, ow=16, ph=8, pw=8,
                    row0=1 * 16 + 1, stride=16, sel=sel2)

        acc3[...] = _conv5x5(xpad3, xcat3, w3, cin=128, wp8=16, m_out=96)
        _relu_store(acc3, b3, xpad4, wp8=16, oh=6, ow=6,
                    row0=1 * 8 + 1, stride=8)

        acc4[...] = _conv5x5(xpad4, xcat4, w4, cin=256, wp8=8, m_out=32)
        _relu_store(acc4, b4, xpad5, wp8=8, oh=4, ow=4,
                    row0=1 * 8 + 1, stride=8)

        a5 = _conv5x5(xpad5, xcat5, w5, cin=256, wp8=8, m_out=16)
        b5v = b5[...]
        for y in range(2):
            for x in range(2):
                row = jnp.maximum(a5[y * 8 + x:y * 8 + x + 1, :] + b5v, 0.0)
                c0 = (y * 2 + x) * 128
                feat[pl.ds(i * 8, 1), c0:c0 + 128] = row.astype(feat.dtype)
        return carry

    jax.lax.fori_loop(0, IB, one_image, 0, unroll=False)

    r = jax.lax.broadcasted_iota(jnp.int32, (IB, IB * 8), 0)
    c = jax.lax.broadcasted_iota(jnp.int32, (IB, IB * 8), 1)
    gather = (c == 8 * r).astype(jnp.bfloat16)
    fblk = jnp.dot(gather, feat[...],
                   preferred_element_type=jnp.float32).astype(jnp.bfloat16)
    h = jnp.dot(fblk, wl1[...],
                preferred_element_type=jnp.float32) + bl1[...]
    h = jnp.dot(h.astype(jnp.bfloat16), wl2[...],
                preferred_element_type=jnp.float32) + bl2[...]
    h = jnp.dot(h.astype(jnp.bfloat16), wl3[...],
                preferred_element_type=jnp.float32) + bl3[...]
    out_ref[...] = 1.0 / (1.0 + jnp.exp(-h))


def _whole(shape):
    return pl.BlockSpec(shape, lambda n: tuple(0 for _ in shape))


def kernel(x, c1_w, c1_b, c2_w, c2_b, c3_w, c3_b, c4_w, c4_b, c5_w, c5_b,
           l1_w, l1_b, l2_w, l2_b, l3_w, l3_b):
    n = x.shape[0]
    xh = jnp.transpose(x, (0, 2, 3, 1)).astype(jnp.bfloat16)
    xh = jnp.pad(xh, ((0, 0), (0, 0), (0, 0), (0, 5)))
    x2d = xh.reshape(n * 1024, 8)

    out = pl.pallas_call(
        _fused_kernel,
        out_shape=jax.ShapeDtypeStruct((n, 128), jnp.float32),
        grid=(n // IB,),
        in_specs=[
            pl.BlockSpec((IB * 1024, 8), lambda n: (n, 0)),
            _whole((5, 40, 128)), _whole((1, 128)),
            _whole((5, 640, 128)), _whole((1, 128)),
            _whole((5, 640, 256)), _whole((1, 256)),
            _whole((5, 1280, 256)), _whole((1, 256)),
            _whole((5, 1280, 128)), _whole((1, 128)),
            _whole((512, 384)), _whole((1, 384)),
            _whole((384, 256)), _whole((1, 256)),
            _whole((256, 128)), _whole((1, 128)),
        ],
        out_specs=pl.BlockSpec((IB, 128), lambda n: (n, 0)),
        scratch_shapes=[
            pltpu.VMEM((1448, 8), jnp.bfloat16),
            pltpu.VMEM((1440, 40), jnp.bfloat16),
            pltpu.VMEM((1280, 128), jnp.float32),
            pltpu.VMEM((488, 128), jnp.bfloat16),
            pltpu.VMEM((480, 640), jnp.bfloat16),
            pltpu.VMEM((384, 128), jnp.float32),
            pltpu.VMEM((168, 128), jnp.bfloat16),
            pltpu.VMEM((160, 640), jnp.bfloat16),
            pltpu.VMEM((96, 256), jnp.float32),
            pltpu.VMEM((72, 256), jnp.bfloat16),
            pltpu.VMEM((64, 1280), jnp.bfloat16),
            pltpu.VMEM((32, 256), jnp.float32),
            pltpu.VMEM((56, 256), jnp.bfloat16),
            pltpu.VMEM((48, 1280), jnp.bfloat16),
            pltpu.VMEM((IB * 8, 512), jnp.bfloat16),
        ],
        compiler_params=pltpu.CompilerParams(
            dimension_semantics=("parallel",),
            vmem_limit_bytes=64 * 1024 * 1024,
        ),
    )(x2d, c1_w, c1_b, c2_w, c2_b, c3_w, c3_b, c4_w, c4_b, c5_w, c5_b,
      l1_w, l1_b, l2_w, l2_b, l3_w, l3_b)
    return out[:, :100]


# speedup vs baseline: 1.1921x; 1.0927x over previous
import jax
import jax.numpy as jnp
from jax.experimental import pallas as pl
from jax.experimental.pallas import tpu as pltpu

IB = 32
N_IMG = 2048


def _even_cols_selector(pw, ow):
    r = jax.lax.broadcasted_iota(jnp.int32, (pw, ow), 0)
    c = jax.lax.broadcasted_iota(jnp.int32, (pw, ow), 1)
    return (c == 2 * r).astype(jnp.float32)


def _conv5x5(xpad, xcat, w_ref, *, cin, wp8, m_out):
    r_cat = xcat.shape[0]
    for j in range(5):
        xcat[:, j * cin:(j + 1) * cin] = xpad[pl.ds(j, r_cat), :]
    acc = None
    for i in range(5):
        part = jnp.dot(xcat[pl.ds(i * wp8, m_out), :], w_ref[i],
                       preferred_element_type=jnp.float32)
        acc = part if acc is None else acc + part
    return acc


def _pool_store(acc, b_ref, dst, *, wp8, ow, ph, pw, row0, stride, sel):
    bias = b_ref[...]
    for py in range(ph):
        r0 = 2 * py * wp8
        m = jnp.maximum(
            jnp.maximum(acc[pl.ds(r0, ow), :], acc[pl.ds(r0 + 1, ow), :]),
            jnp.maximum(acc[pl.ds(r0 + wp8, ow), :],
                        acc[pl.ds(r0 + wp8 + 1, ow), :]))
        p = jnp.dot(sel, m, preferred_element_type=jnp.float32)
        p = jnp.maximum(p + bias, 0.0).astype(dst.dtype)
        dst[pl.ds(row0 + py * stride, pw), :] = p


def _relu_store(acc, b_ref, dst, *, wp8, oh, ow, row0, stride):
    bias = b_ref[...]
    for y in range(oh):
        v = jnp.maximum(acc[pl.ds(y * wp8, ow), :] + bias, 0.0)
        dst[pl.ds(row0 + y * stride, ow), :] = v.astype(dst.dtype)


def _image_pipeline(i, x_ref, w1, b1, w2, b2, w3, b3, w4, b4, w5, b5,
                    feat, sel1, sel2, bufs):
    (xpad1, xcat1, acc1, xpad2, xcat2, acc2, xpad3, xcat3, acc3,
     xpad4, xcat4, acc4, xpad5, xcat5) = bufs
    base = i * 1024
    for y in range(32):
        xpad1[pl.ds((2 + y) * 40 + 2, 32), :] = \
            x_ref[pl.ds(base + y * 32, 32), :]

    acc1[...] = _conv5x5(xpad1, xcat1, w1, cin=8, wp8=40, m_out=1280)
    _pool_store(acc1, b1, xpad2, wp8=40, ow=32, ph=16, pw=16,
                row0=2 * 24 + 2, stride=24, sel=sel1)

    acc2[...] = _conv5x5(xpad2, xcat2, w2, cin=128, wp8=24, m_out=384)
    _pool_store(acc2, b2, xpad3, wp8=24, ow=16, ph=8, pw=8,
                row0=1 * 16 + 1, stride=16, sel=sel2)

    acc3[...] = _conv5x5(xpad3, xcat3, w3, cin=128, wp8=16, m_out=96)
    _relu_store(acc3, b3, xpad4, wp8=16, oh=6, ow=6,
                row0=1 * 8 + 1, stride=8)

    acc4[...] = _conv5x5(xpad4, xcat4, w4, cin=256, wp8=8, m_out=32)
    _relu_store(acc4, b4, xpad5, wp8=8, oh=4, ow=4,
                row0=1 * 8 + 1, stride=8)

    a5 = _conv5x5(xpad5, xcat5, w5, cin=256, wp8=8, m_out=16)
    b5v = b5[...]
    for y in range(2):
        for x in range(2):
            row = jnp.maximum(a5[y * 8 + x:y * 8 + x + 1, :] + b5v, 0.0)
            c0 = (y * 2 + x) * 128
            feat[pl.ds(i * 8, 1), c0:c0 + 128] = row.astype(feat.dtype)


def _fused_kernel(x_ref, w1, b1, w2, b2, w3, b3, w4, b4, w5, b5,
                  wl1, bl1, wl2, bl2, wl3, bl3, out_ref, feat, *scratch):
    bufs_a = scratch[:14]
    bufs_b = scratch[14:]
    for bufs in (bufs_a, bufs_b):
        for idx in (0, 3, 6, 9, 12):
            bufs[idx][...] = jnp.zeros_like(bufs[idx])
    feat[...] = jnp.zeros_like(feat)

    sel1 = _even_cols_selector(16, 32)
    sel2 = _even_cols_selector(8, 16)

    def two_images(j, carry):
        args = (x_ref, w1, b1, w2, b2, w3, b3, w4, b4, w5, b5,
                feat, sel1, sel2)
        _image_pipeline(2 * j, *args, bufs_a)
        _image_pipeline(2 * j + 1, *args, bufs_b)
        return carry

    jax.lax.fori_loop(0, IB // 2, two_images, 0, unroll=False)

    r = jax.lax.broadcasted_iota(jnp.int32, (IB, IB * 8), 0)
    c = jax.lax.broadcasted_iota(jnp.int32, (IB, IB * 8), 1)
    gather = (c == 8 * r).astype(jnp.float32)
    fblk = jnp.dot(gather, feat[...],
                   preferred_element_type=jnp.float32).astype(jnp.bfloat16)
    h = jnp.dot(fblk, wl1[...],
                preferred_element_type=jnp.float32) + bl1[...]
    h = jnp.dot(h.astype(jnp.bfloat16), wl2[...],
                preferred_element_type=jnp.float32) + bl2[...]
    h = jnp.dot(h.astype(jnp.bfloat16), wl3[...],
                preferred_element_type=jnp.float32) + bl3[...]
    out_ref[...] = 1.0 / (1.0 + jnp.exp(-h))


def _whole(shape):
    return pl.BlockSpec(shape, lambda n: tuple(0 for _ in shape))


def _conv_bufs():
    return [
        pltpu.VMEM((1448, 8), jnp.bfloat16),
        pltpu.VMEM((1440, 40), jnp.bfloat16),
        pltpu.VMEM((1280, 128), jnp.float32),
        pltpu.VMEM((488, 128), jnp.bfloat16),
        pltpu.VMEM((480, 640), jnp.bfloat16),
        pltpu.VMEM((384, 128), jnp.float32),
        pltpu.VMEM((168, 128), jnp.bfloat16),
        pltpu.VMEM((160, 640), jnp.bfloat16),
        pltpu.VMEM((96, 256), jnp.float32),
        pltpu.VMEM((72, 256), jnp.bfloat16),
        pltpu.VMEM((64, 1280), jnp.bfloat16),
        pltpu.VMEM((32, 256), jnp.float32),
        pltpu.VMEM((56, 256), jnp.bfloat16),
        pltpu.VMEM((48, 1280), jnp.bfloat16),
    ]


def kernel(x, c1_w, c1_b, c2_w, c2_b, c3_w, c3_b, c4_w, c4_b, c5_w, c5_b,
           l1_w, l1_b, l2_w, l2_b, l3_w, l3_b):
    n = x.shape[0]
    xh = jnp.transpose(x, (0, 2, 3, 1)).astype(jnp.bfloat16)
    xh = jnp.pad(xh, ((0, 0), (0, 0), (0, 0), (0, 5)))
    x2d = xh.reshape(n * 1024, 8)

    out = pl.pallas_call(
        _fused_kernel,
        out_shape=jax.ShapeDtypeStruct((n, 128), jnp.float32),
        grid=(n // IB,),
        in_specs=[
            pl.BlockSpec((IB * 1024, 8), lambda n: (n, 0)),
            _whole((5, 40, 128)), _whole((1, 128)),
            _whole((5, 640, 128)), _whole((1, 128)),
            _whole((5, 640, 256)), _whole((1, 256)),
            _whole((5, 1280, 256)), _whole((1, 256)),
            _whole((5, 1280, 128)), _whole((1, 128)),
            _whole((512, 384)), _whole((1, 384)),
            _whole((384, 256)), _whole((1, 256)),
            _whole((256, 128)), _whole((1, 128)),
        ],
        out_specs=pl.BlockSpec((IB, 128), lambda n: (n, 0)),
        scratch_shapes=[pltpu.VMEM((IB * 8, 512), jnp.float32)]
        + _conv_bufs() + _conv_bufs(),
        compiler_params=pltpu.CompilerParams(
            dimension_semantics=("parallel",),
            vmem_limit_bytes=64 * 1024 * 1024,
        ),
    )(x2d, c1_w, c1_b, c2_w, c2_b, c3_w, c3_b, c4_w, c4_b, c5_w, c5_b,
      l1_w, l1_b, l2_w, l2_b, l3_w, l3_b)
    return out[:, :100]


# speedup vs baseline: 1.6553x; 1.3886x over previous
import jax
import jax.numpy as jnp
from jax.experimental import pallas as pl
from jax.experimental.pallas import tpu as pltpu

IB = 32
BL = 128


def _even_cols_selector(pw, ow):
    r = jax.lax.broadcasted_iota(jnp.int32, (pw, ow), 0)
    c = jax.lax.broadcasted_iota(jnp.int32, (pw, ow), 1)
    return (c == 2 * r).astype(jnp.float32)


def _conv5x5(xpad, xcat, w_ref, *, cin, wp8, m_out):
    r_cat = xcat.shape[0]
    for j in range(5):
        xcat[:, j * cin:(j + 1) * cin] = xpad[pl.ds(j, r_cat), :]
    acc = None
    for i in range(5):
        part = jnp.dot(xcat[pl.ds(i * wp8, m_out), :], w_ref[i],
                       preferred_element_type=jnp.float32)
        acc = part if acc is None else acc + part
    return acc


def _pool_store(acc, b_ref, dst, *, wp8, ow, ph, pw, row0, stride, sel):
    bias = b_ref[...]
    for py in range(ph):
        r0 = 2 * py * wp8
        m = jnp.maximum(
            jnp.maximum(acc[pl.ds(r0, ow), :], acc[pl.ds(r0 + 1, ow), :]),
            jnp.maximum(acc[pl.ds(r0 + wp8, ow), :],
                        acc[pl.ds(r0 + wp8 + 1, ow), :]))
        p = jnp.dot(sel, m, preferred_element_type=jnp.float32)
        p = jnp.maximum(p + bias, 0.0).astype(dst.dtype)
        dst[pl.ds(row0 + py * stride, pw), :] = p


def _head_pipeline(i, x_ref, w1, b1, w2, b2, out_ref, sel1, sel2, bufs):
    xpad1, xcat1, acc1, xpad2, xcat2, acc2 = bufs
    base = i * 1024
    for y in range(32):
        xpad1[pl.ds((2 + y) * 40 + 2, 32), :] = \
            x_ref[pl.ds(base + y * 32, 32), :]

    acc1[...] = _conv5x5(xpad1, xcat1, w1, cin=8, wp8=40, m_out=1280)
    _pool_store(acc1, b1, xpad2, wp8=40, ow=32, ph=16, pw=16,
                row0=2 * 24 + 2, stride=24, sel=sel1)

    acc2[...] = _conv5x5(xpad2, xcat2, w2, cin=128, wp8=24, m_out=384)
    _pool_store(acc2, b2, out_ref, wp8=24, ow=16, ph=8, pw=8,
                row0=i * 64, stride=8, sel=sel2)


def _head_kernel(x_ref, w1, b1, w2, b2, out_ref, *scratch):
    bufs_a = scratch[:6]
    bufs_b = scratch[6:]
    for bufs in (bufs_a, bufs_b):
        bufs[0][...] = jnp.zeros_like(bufs[0])
        bufs[3][...] = jnp.zeros_like(bufs[3])

    sel1 = _even_cols_selector(16, 32)
    sel2 = _even_cols_selector(8, 16)

    def two_images(j, carry):
        _head_pipeline(2 * j, x_ref, w1, b1, w2, b2, out_ref,
                       sel1, sel2, bufs_a)
        _head_pipeline(2 * j + 1, x_ref, w1, b1, w2, b2, out_ref,
                       sel1, sel2, bufs_b)
        return carry

    jax.lax.fori_loop(0, IB // 2, two_images, 0, unroll=False)


def _cb_conv(w_ref, b_ref, src, dst, *, cin, cout, iw, oh, ow, dw, store):
    bias = b_ref[...]
    for y in range(oh):
        for x in range(ow):
            acc = None
            for kh in range(5):
                r0 = ((y + kh) * iw + x) * cin
                part = jnp.dot(w_ref[kh], src[pl.ds(r0, 5 * cin), :],
                               preferred_element_type=jnp.float32)
                acc = part if acc is None else acc + part
            v = jnp.maximum(acc + bias, 0.0)
            store(y * ow + x, v)
    _ = dw


def _tail_kernel(x_ref, w3, b3, w4, b4, w5, b5,
                 wl1, bl1, wl2, bl2, wl3, bl3, out_ref,
                 xp3, xp4, xp5, feat):
    @pl.when(pl.program_id(1) == 0)
    def _zero():
        xp3[...] = jnp.zeros_like(xp3)
        xp4[...] = jnp.zeros_like(xp4)
        xp5[...] = jnp.zeros_like(xp5)

    for y in range(8):
        xp3[pl.ds(((y + 1) * 10 + 1) * 128, 1024), :] = \
            x_ref[pl.ds(y * 1024, 1024), :]

    def store4(pos, v):
        y, x = divmod(pos, 6)
        r = ((y + 1) * 8 + (x + 1)) * 256
        xp4[pl.ds(r, 256), :] = v.astype(jnp.bfloat16)

    _cb_conv(w3, b3, xp3, xp4, cin=128, cout=256, iw=10, oh=6, ow=6,
             dw=8, store=store4)

    def store5(pos, v):
        y, x = divmod(pos, 4)
        r = ((y + 1) * 6 + (x + 1)) * 256
        xp5[pl.ds(r, 256), :] = v.astype(jnp.bfloat16)

    _cb_conv(w4, b4, xp4, xp5, cin=256, cout=256, iw=8, oh=4, ow=4,
             dw=6, store=store5)

    def storef(pos, v):
        feat[pl.ds(pos * 128, 128), :] = v.astype(jnp.bfloat16)

    _cb_conv(w5, b5, xp5, feat, cin=256, cout=128, iw=6, oh=2, ow=2,
             dw=0, store=storef)

    h = jnp.dot(wl1[...], feat[...],
                preferred_element_type=jnp.float32) + bl1[...]
    h = jnp.dot(wl2[...], h.astype(jnp.bfloat16),
                preferred_element_type=jnp.float32) + bl2[...]
    h = jnp.dot(wl3[...], h.astype(jnp.bfloat16),
                preferred_element_type=jnp.float32) + bl3[...]
    out_ref[...] = 1.0 / (1.0 + jnp.exp(-h))


def _whole(shape):
    return pl.BlockSpec(shape, lambda *g: tuple(0 for _ in shape))


def kernel(x, c1_w, c1_b, c2_w, c2_b, c3_w, c3_b, c4_w, c4_b, c5_w, c5_b,
           l1_w, l1_b, l2_w, l2_b, l3_w, l3_b):
    n = x.shape[0]
    xh = jnp.transpose(x, (0, 2, 3, 1)).astype(jnp.bfloat16)
    xh = jnp.pad(xh, ((0, 0), (0, 0), (0, 0), (0, 5)))
    x2d = xh.reshape(n * 1024, 8)

    pooled = pl.pallas_call(
        _head_kernel,
        out_shape=jax.ShapeDtypeStruct((n * 64, 128), jnp.bfloat16),
        grid=(n // IB,),
        in_specs=[
            pl.BlockSpec((IB * 1024, 8), lambda g: (g, 0)),
            _whole((5, 40, 128)), _whole((1, 128)),
            _whole((5, 640, 128)), _whole((1, 128)),
        ],
        out_specs=pl.BlockSpec((IB * 64, 128), lambda g: (g, 0)),
        scratch_shapes=[
            pltpu.VMEM((1448, 8), jnp.bfloat16),
            pltpu.VMEM((1440, 40), jnp.bfloat16),
            pltpu.VMEM((1280, 128), jnp.float32),
            pltpu.VMEM((488, 128), jnp.bfloat16),
            pltpu.VMEM((480, 640), jnp.bfloat16),
            pltpu.VMEM((384, 128), jnp.float32),
        ] * 2,
        compiler_params=pltpu.CompilerParams(
            dimension_semantics=("parallel",),
            vmem_limit_bytes=64 * 1024 * 1024,
        ),
    )(x2d, c1_w, c1_b, c2_w, c2_b)

    nb = n // BL
    pcb = jnp.swapaxes(pooled.reshape(nb, BL, 64 * 128), 1, 2)
    pcb = pcb.reshape(nb * 64 * 128, BL)

    w3t = jnp.swapaxes(c3_w, 1, 2)
    w4t = jnp.swapaxes(c4_w, 1, 2)
    w5t = jnp.swapaxes(c5_w, 1, 2)

    out = pl.pallas_call(
        _tail_kernel,
        out_shape=jax.ShapeDtypeStruct((nb * 128, BL), jnp.float32),
        grid=(2, nb // 2),
        in_specs=[
            pl.BlockSpec((64 * 128, BL), lambda c, j: (c * (nb // 2) + j, 0)),
            _whole((5, 256, 640)), _whole((256, 1)),
            _whole((5, 256, 1280)), _whole((256, 1)),
            _whole((5, 128, 1280)), _whole((128, 1)),
            _whole((384, 512)), _whole((384, 1)),
            _whole((256, 384)), _whole((256, 1)),
            _whole((128, 256)), _whole((128, 1)),
        ],
        out_specs=pl.BlockSpec((128, BL),
                               lambda c, j: (c * (nb // 2) + j, 0)),
        scratch_shapes=[
            pltpu.VMEM((10 * 10 * 128, BL), jnp.bfloat16),
            pltpu.VMEM((8 * 8 * 256, BL), jnp.bfloat16),
            pltpu.VMEM((6 * 6 * 256, BL), jnp.bfloat16),
            pltpu.VMEM((512, BL), jnp.bfloat16),
        ],
        compiler_params=pltpu.CompilerParams(
            dimension_semantics=("parallel", "arbitrary"),
            vmem_limit_bytes=64 * 1024 * 1024,
        ),
    )(pcb, w3t, c3_b.reshape(256, 1), w4t, c4_b.reshape(256, 1),
      w5t, c5_b.reshape(128, 1), l1_w.T, l1_b.reshape(384, 1),
      l2_w.T, l2_b.reshape(256, 1), l3_w.T, l3_b.reshape(128, 1))
    out = jnp.swapaxes(out.reshape(nb, 128, BL), 1, 2).reshape(n, 128)
    return out[:, :100]
